# baseline (speedup 1.0000x reference)
import jax
import jax.numpy as jnp
import numpy as np
from jax import lax
from jax.experimental import pallas as pl
from jax.experimental.pallas import tpu as pltpu
from jax.experimental.pallas import tpu_sc as plsc

_NX = 128
_NY = 128
_NCLS = 3
_NB = 1000
_NBP = 1008
_NCHUNK = _NBP // 16
_NW = 32
_ROWS = _NX // _NW
_CLS_W = _NCLS * _ROWS * _NY * 5
_REG_W = _NCLS * _ROWS * _NY * 7

_W0, _W12 = 1.6, 0.6
_L0, _L1, _L2 = 3.9, 0.8, 1.76
_H0, _H12 = 1.56, 1.73
_CZ0, _CZ12 = -1.0, -0.6
_T0, _T12 = 0.6, 0.35
_YAW1 = 1.5708


_NEG_CACHE = None


def _neg_base():
    key = jax.random.key(1)
    masks = []
    for _ in range(_NCLS):
        key, k1, k2 = jax.random.split(key, 3)
        ni = np.asarray(jax.random.randint(k1, (5000,), 0, _NX))
        nj = np.asarray(jax.random.randint(k2, (5000,), 0, _NY))
        m = np.zeros((_NX, _NY), np.bool_)
        m[ni, nj] = True
        masks.append(m)
    neg = np.stack(masks)
    base = np.zeros((_NCLS, _NX, _NY, 5), np.int32)
    base[..., 3] = neg
    base[..., 4] = ~neg
    return np.ascontiguousarray(
        base.reshape(_NCLS, _NW, _ROWS, _NY, 5).transpose(1, 0, 2, 3, 4)
    ).reshape(_NW, _CLS_W)


def _get_neg_base():
    global _NEG_CACHE
    if _NEG_CACHE is None:
        with jax.ensure_compile_time_eval():
            _NEG_CACHE = _neg_base()
    return _NEG_CACHE


def _body(fb, fc, base_h,
          ocls, oreg,
          vf, vc,
          cand, clsl, regl, sem):
    f32 = jnp.float32
    i32 = jnp.int32
    wid = lax.axis_index("s") * 2 + lax.axis_index("c")

    copies = [pltpu.async_copy(fb, vf, sem),
              pltpu.async_copy(fc, vc, sem),
              pltpu.async_copy(base_h.at[wid], clsl, sem)]

    zero16 = jnp.zeros((16,), jnp.float32)

    def zbody(k, carry):
        for u in range(8):
            regl[pl.ds((k * 8 + u) * 16, 16)] = zero16
        return carry

    lax.fori_loop(0, _REG_W // 128, zbody, 0)
    for cdesc in copies:
        cdesc.wait()

    iota = jnp.arange(16, dtype=i32)
    widv = jnp.full((16,), wid, i32).astype(f32)
    cx_lo = -31.75 + 2.0 * widv
    cx_hi = cx_lo + 1.5

    def fbody(k, cnt):
        sl = pl.ds(k * 16, 16)
        bxv = vf[0, sl]
        bwv = vf[3, sl]
        blv = vf[4, sl]
        cv = vc[sl]
        wv = jnp.where(cv == 0, _W0, _W12)
        lv = jnp.where(cv == 0, _L0, jnp.where(cv == 1, _L1, _L2))
        thrv = jnp.where(cv == 0, _T0, _T12)
        hw = (bwv + wv) * 0.5
        m = (bxv + hw > cx_lo) & (bxv - hw < cx_hi)
        imax = jnp.minimum(bwv, wv) * jnp.minimum(blv, lv)
        m &= imax > (0.999 * thrv) * (wv * lv + bwv * blv - imax)
        cum = jnp.cumsum(m.astype(i32))
        plsc.store_scatter(cand, [cnt + cum - 1], k * 16 + iota, mask=m)
        return cnt + jnp.max(cum)

    ncand = lax.fori_loop(0, _NCHUNK, fbody, i32(0))

    def cbody(t, carry):
        b = plsc.load_gather(cand, [jnp.full((16,), t, i32)])
        zidx = jnp.zeros((16,), i32)
        bxv = plsc.load_gather(vf, [zidx, b])
        byv = plsc.load_gather(vf, [zidx + 1, b])
        bzv = plsc.load_gather(vf, [zidx + 2, b])
        bwv = plsc.load_gather(vf, [zidx + 3, b])
        blv = plsc.load_gather(vf, [zidx + 4, b])
        bhv = plsc.load_gather(vf, [zidx + 5, b])
        bywv = plsc.load_gather(vf, [zidx + 6, b])
        cv = plsc.load_gather(vc, [b])

        wv = jnp.where(cv == 0, _W0, _W12)
        lv = jnp.where(cv == 0, _L0, jnp.where(cv == 1, _L1, _L2))
        hv = jnp.where(cv == 0, _H0, _H12)
        czv = jnp.where(cv == 0, _CZ0, _CZ12)
        thrv = jnp.where(cv == 0, _T0, _T12)
        whv = wv * 0.5
        lhv = lv * 0.5
        area = wv * lv

        lo = (byv - (blv + lv) * 0.5 + 31.75) * 2.0
        jv = (lo + 16.0).astype(i32) - 15 + iota
        valid = (jv >= 0) & (jv < _NY)
        jsafe = jnp.clip(jv, 0, _NY - 1)
        cyv = -31.75 + 0.5 * jv.astype(f32)
        iyv = jnp.maximum(
            jnp.minimum(cyv + lhv, byv + blv * 0.5)
            - jnp.maximum(cyv - lhv, byv - blv * 0.5), 0.0)

        r1 = byv - cyv
        r2 = bzv - czv
        r3 = (bwv - wv) / wv
        r4 = (blv - lv) / lv
        r5 = (bhv - hv) / hv
        r6 = bywv - _YAW1
        c0v = jnp.where(cv == 0, 1, 0).astype(i32)
        c1v = jnp.where(cv == 1, 1, 0).astype(i32)
        c2v = jnp.where(cv == 2, 1, 0).astype(i32)
        zi = jnp.zeros((16,), i32)
        clsbase = cv * (_ROWS * _NY * 5) + jsafe * 5
        regbase = cv * (_ROWS * _NY * 7) + jsafe * 7

        for il in range(_ROWS):
            cxv = cx_lo + (0.5 * il)
            ixv = jnp.maximum(
                jnp.minimum(cxv + whv, bxv + bwv * 0.5)
                - jnp.maximum(cxv - whv, bxv - bwv * 0.5), 0.0)
            inter = ixv * iyv
            union = area + bwv * blv - inter
            iou = inter / jnp.maximum(union, 1e-6)
            posm = (iou > thrv) & valid

            @pl.when(jnp.any(posm))
            def _write(il=il, posm=posm, cxv=cxv):
                cb = clsbase + il * (_NY * 5)
                plsc.store_scatter(clsl, [cb], c0v, mask=posm)
                plsc.store_scatter(clsl, [cb + 1], c1v, mask=posm)
                plsc.store_scatter(clsl, [cb + 2], c2v, mask=posm)
                plsc.store_scatter(clsl, [cb + 3], zi, mask=posm)
                plsc.store_scatter(clsl, [cb + 4], zi, mask=posm)
                rb = regbase + il * (_NY * 7)
                plsc.store_scatter(regl, [rb], bxv - cxv, mask=posm)
                plsc.store_scatter(regl, [rb + 1], r1, mask=posm)
                plsc.store_scatter(regl, [rb + 2], r2, mask=posm)
                plsc.store_scatter(regl, [rb + 3], r3, mask=posm)
                plsc.store_scatter(regl, [rb + 4], r4, mask=posm)
                plsc.store_scatter(regl, [rb + 5], r5, mask=posm)
                plsc.store_scatter(regl, [rb + 6], r6, mask=posm)
        return carry

    lax.fori_loop(0, ncand, cbody, 0)

    pltpu.sync_copy(clsl, ocls.at[wid])
    pltpu.sync_copy(regl, oreg.at[wid])


_assign = pl.kernel(
    _body,
    out_type=(jax.ShapeDtypeStruct((_NW, _CLS_W), jnp.int32),
              jax.ShapeDtypeStruct((_NW, _REG_W), jnp.float32)),
    mesh=plsc.VectorSubcoreMesh(core_axis_name="c", subcore_axis_name="s",
                                num_cores=2, num_subcores=16),
    compiler_params=pltpu.CompilerParams(needs_layout_passes=False),
    scratch_types=[
        pltpu.VMEM((7, _NBP), jnp.float32),
        pltpu.VMEM((_NBP,), jnp.int32),
        pltpu.VMEM((1024,), jnp.int32),
        pltpu.VMEM((_CLS_W,), jnp.int32),
        pltpu.VMEM((_REG_W,), jnp.float32),
        pltpu.SemaphoreType.DMA,
    ],
)


def kernel(boxes, class_idx):
    f32 = jnp.float32
    pad = jnp.full((7, _NBP - _NB), 1.0, f32).at[0].set(1e9)
    fb = jnp.concatenate([boxes.T, pad], axis=1)
    fc = jnp.concatenate([class_idx.astype(jnp.int32),
                          jnp.zeros((_NBP - _NB,), jnp.int32)])
    base = jnp.asarray(_get_neg_base())
    cls_f, reg_f = _assign(fb, fc, base)
    cls = cls_f.reshape(_NCLS, _NX, _NY, 5)
    reg = reg_f.reshape(_NCLS, _NX, _NY, 7)
    return cls, reg


# speedup vs baseline: 2.5897x; 2.5897x over previous
import jax
import jax.numpy as jnp
import numpy as np
from jax import lax
from jax.experimental import pallas as pl
from jax.experimental.pallas import tpu as pltpu
from jax.experimental.pallas import tpu_sc as plsc

_NX = 128
_NY = 128
_NCLS = 3
_NB = 1000
_NBP = 1008
_NCHUNK = _NBP // 16
_NW = 32
_ROWS = _NX // _NW
_CLS_W = _NCLS * _ROWS * _NY * 5
_REG_W = _NCLS * _ROWS * _NY * 7

_W0, _W12 = 1.6, 0.6
_L0, _L1, _L2 = 3.9, 0.8, 1.76
_H0, _H12 = 1.56, 1.73
_CZ0, _CZ12 = -1.0, -0.6
_T0, _T12 = 0.6, 0.35
_YAW1 = 1.5708


_NEG_CACHE = None


def _neg_base():
    key = jax.random.key(1)
    masks = []
    for _ in range(_NCLS):
        key, k1, k2 = jax.random.split(key, 3)
        ni = np.asarray(jax.random.randint(k1, (5000,), 0, _NX))
        nj = np.asarray(jax.random.randint(k2, (5000,), 0, _NY))
        m = np.zeros((_NX, _NY), np.bool_)
        m[ni, nj] = True
        masks.append(m)
    neg = np.stack(masks)
    base = np.zeros((_NCLS, _NX, _NY, 5), np.int32)
    base[..., 3] = neg
    base[..., 4] = ~neg
    return np.ascontiguousarray(
        base.reshape(_NCLS, _NW, _ROWS, _NY, 5).transpose(1, 0, 2, 3, 4)
    ).reshape(_NW, _CLS_W)


def _get_neg_base():
    global _NEG_CACHE
    if _NEG_CACHE is None:
        with jax.ensure_compile_time_eval():
            _NEG_CACHE = _neg_base()
    return _NEG_CACHE


def _body(fb, fc, base_h,
          oall,
          vf, vc,
          cand, clsl, regl, sem):
    f32 = jnp.float32
    i32 = jnp.int32
    wid = lax.axis_index("s") * 2 + lax.axis_index("c")

    copies = [pltpu.async_copy(fb, vf, sem),
              pltpu.async_copy(fc, vc, sem),
              pltpu.async_copy(base_h.at[wid], clsl, sem)]

    zero16 = jnp.zeros((16,), jnp.int32)

    def zbody(k, carry):
        for u in range(8):
            regl[pl.ds((k * 8 + u) * 16, 16)] = zero16
        return carry

    lax.fori_loop(0, _REG_W // 128, zbody, 0)
    for cdesc in copies:
        cdesc.wait()

    iota = jnp.arange(16, dtype=i32)
    widv = jnp.full((16,), wid, i32).astype(f32)
    cx_lo = -31.75 + 2.0 * widv
    cx_hi = cx_lo + 1.5

    def fbody(k, cnt):
        sl = pl.ds(k * 16, 16)
        bxv = vf[0, sl]
        bwv = vf[3, sl]
        blv = vf[4, sl]
        cv = vc[sl]
        wv = jnp.where(cv == 0, _W0, _W12)
        lv = jnp.where(cv == 0, _L0, jnp.where(cv == 1, _L1, _L2))
        thrv = jnp.where(cv == 0, _T0, _T12)
        hw = (bwv + wv) * 0.5
        m = (bxv + hw > cx_lo) & (bxv - hw < cx_hi)
        imax = jnp.minimum(bwv, wv) * jnp.minimum(blv, lv)
        m &= imax > (0.999 * thrv) * (wv * lv + bwv * blv - imax)
        cum = jnp.cumsum(m.astype(i32))
        plsc.store_scatter(cand, [cnt + cum - 1], k * 16 + iota, mask=m)
        return cnt + jnp.max(cum)

    ncand = lax.fori_loop(0, _NCHUNK, fbody, i32(0))

    def cbody(t, carry):
        b = plsc.load_gather(cand, [jnp.full((16,), t, i32)])
        zidx = jnp.zeros((16,), i32)
        bxv = plsc.load_gather(vf, [zidx, b])
        byv = plsc.load_gather(vf, [zidx + 1, b])
        bzv = plsc.load_gather(vf, [zidx + 2, b])
        bwv = plsc.load_gather(vf, [zidx + 3, b])
        blv = plsc.load_gather(vf, [zidx + 4, b])
        bhv = plsc.load_gather(vf, [zidx + 5, b])
        bywv = plsc.load_gather(vf, [zidx + 6, b])
        cv = plsc.load_gather(vc, [b])

        wv = jnp.where(cv == 0, _W0, _W12)
        lv = jnp.where(cv == 0, _L0, jnp.where(cv == 1, _L1, _L2))
        hv = jnp.where(cv == 0, _H0, _H12)
        czv = jnp.where(cv == 0, _CZ0, _CZ12)
        thrv = jnp.where(cv == 0, _T0, _T12)
        whv = wv * 0.5
        lhv = lv * 0.5
        area = wv * lv

        lo = (byv - (blv + lv) * 0.5 + 31.75) * 2.0
        jv = (lo + 16.0).astype(i32) - 15 + iota
        valid = (jv >= 0) & (jv < _NY)
        jsafe = jnp.clip(jv, 0, _NY - 1)
        cyv = -31.75 + 0.5 * jv.astype(f32)
        iyv = jnp.maximum(
            jnp.minimum(cyv + lhv, byv + blv * 0.5)
            - jnp.maximum(cyv - lhv, byv - blv * 0.5), 0.0)

        r1 = byv - cyv
        r2 = bzv - czv
        r3 = (bwv - wv) / wv
        r4 = (blv - lv) / lv
        r5 = (bhv - hv) / hv
        r6 = bywv - _YAW1
        c0v = jnp.where(cv == 0, 1, 0).astype(i32)
        c1v = jnp.where(cv == 1, 1, 0).astype(i32)
        c2v = jnp.where(cv == 2, 1, 0).astype(i32)
        zi = jnp.zeros((16,), i32)
        clsbase = cv * (_ROWS * _NY * 5) + jsafe * 5
        regbase = cv * (_ROWS * _NY * 7) + jsafe * 7

        for il in range(_ROWS):
            cxv = cx_lo + (0.5 * il)
            ixv = jnp.maximum(
                jnp.minimum(cxv + whv, bxv + bwv * 0.5)
                - jnp.maximum(cxv - whv, bxv - bwv * 0.5), 0.0)
            inter = ixv * iyv
            union = area + bwv * blv - inter
            iou = inter / jnp.maximum(union, 1e-6)
            posm = (iou > thrv) & valid

            @pl.when(jnp.any(posm))
            def _write(il=il, posm=posm, cxv=cxv):
                cb = clsbase + il * (_NY * 5)
                plsc.store_scatter(clsl, [cb], c0v, mask=posm)
                plsc.store_scatter(clsl, [cb + 1], c1v, mask=posm)
                plsc.store_scatter(clsl, [cb + 2], c2v, mask=posm)
                plsc.store_scatter(clsl, [cb + 3], zi, mask=posm)
                plsc.store_scatter(clsl, [cb + 4], zi, mask=posm)
                rb = regbase + il * (_NY * 7)
                plsc.store_scatter(regl, [rb], plsc.bitcast(bxv - cxv, jnp.int32), mask=posm)
                plsc.store_scatter(regl, [rb + 1], plsc.bitcast(r1, jnp.int32), mask=posm)
                plsc.store_scatter(regl, [rb + 2], plsc.bitcast(r2, jnp.int32), mask=posm)
                plsc.store_scatter(regl, [rb + 3], plsc.bitcast(r3, jnp.int32), mask=posm)
                plsc.store_scatter(regl, [rb + 4], plsc.bitcast(r4, jnp.int32), mask=posm)
                plsc.store_scatter(regl, [rb + 5], plsc.bitcast(r5, jnp.int32), mask=posm)
                plsc.store_scatter(regl, [rb + 6], plsc.bitcast(r6, jnp.int32), mask=posm)
        return carry

    lax.fori_loop(0, ncand, cbody, 0)

    pltpu.sync_copy(clsl, oall.at[wid, pl.ds(0, _CLS_W)])
    pltpu.sync_copy(regl, oall.at[wid, pl.ds(_CLS_W, _REG_W)])


_assign = pl.kernel(
    _body,
    out_type=jax.ShapeDtypeStruct((_NW, _CLS_W + _REG_W), jnp.int32),
    mesh=plsc.VectorSubcoreMesh(core_axis_name="c", subcore_axis_name="s",
                                num_cores=2, num_subcores=16),
    compiler_params=pltpu.CompilerParams(needs_layout_passes=False),
    scratch_types=[
        pltpu.VMEM((7, _NBP), jnp.float32),
        pltpu.VMEM((_NBP,), jnp.int32),
        pltpu.VMEM((1024,), jnp.int32),
        pltpu.VMEM((_CLS_W,), jnp.int32),
        pltpu.VMEM((_REG_W,), jnp.int32),
        pltpu.SemaphoreType.DMA,
    ],
)


def kernel(boxes, class_idx):
    f32 = jnp.float32
    pad = jnp.full((7, _NBP - _NB), 1.0, f32).at[0].set(1e9)
    fb = jnp.concatenate([boxes.T, pad], axis=1)
    fc = jnp.concatenate([class_idx.astype(jnp.int32),
                          jnp.zeros((_NBP - _NB,), jnp.int32)])
    base = jnp.asarray(_get_neg_base())
    all_f = _assign(fb, fc, base)
    cls_f = all_f[:, :_CLS_W]
    reg_f = lax.bitcast_convert_type(all_f[:, _CLS_W:], jnp.float32)
    cls = (cls_f.reshape(_NW, _NCLS, _ROWS, _NY, 5)
           .transpose(1, 0, 2, 3, 4).reshape(_NCLS, _NX, _NY, 5))
    reg = (reg_f.reshape(_NW, _NCLS, _ROWS, _NY, 7)
           .transpose(1, 0, 2, 3, 4).reshape(_NCLS, _NX, _NY, 7))
    return cls, reg


# speedup vs baseline: 2.6787x; 1.0344x over previous
import jax
import jax.numpy as jnp
import numpy as np
from jax import lax
from jax.experimental import pallas as pl
from jax.experimental.pallas import tpu as pltpu
from jax.experimental.pallas import tpu_sc as plsc

_NX = 128
_NY = 128
_NCLS = 3
_NB = 1000
_NBP = 1008
_NCHUNK = _NBP // 16
_NW = 32
_ROWS = _NX // _NW
_CLS_W = _NCLS * _ROWS * _NY * 5
_REG_W = _NCLS * _ROWS * _NY * 7

_W0, _W12 = 1.6, 0.6
_L0, _L1, _L2 = 3.9, 0.8, 1.76
_H0, _H12 = 1.56, 1.73
_CZ0, _CZ12 = -1.0, -0.6
_T0, _T12 = 0.6, 0.35
_YAW1 = 1.5708


_NEG_CACHE = None


def _neg_base():
    key = jax.random.key(1)
    masks = []
    for _ in range(_NCLS):
        key, k1, k2 = jax.random.split(key, 3)
        ni = np.asarray(jax.random.randint(k1, (5000,), 0, _NX))
        nj = np.asarray(jax.random.randint(k2, (5000,), 0, _NY))
        m = np.zeros((_NX, _NY), np.bool_)
        m[ni, nj] = True
        masks.append(m)
    neg = np.stack(masks)
    base = np.zeros((_NCLS, _NX, _NY, 5), np.int32)
    base[..., 3] = neg
    base[..., 4] = ~neg
    return np.ascontiguousarray(
        base.reshape(_NCLS, _NW, _ROWS, _NY, 5).transpose(1, 0, 2, 3, 4)
    ).reshape(_NW, _CLS_W)


def _get_neg_base():
    global _NEG_CACHE
    if _NEG_CACHE is None:
        with jax.ensure_compile_time_eval():
            _NEG_CACHE = _neg_base()
    return _NEG_CACHE


def _body(fb, fc, base_h,
          ocls, oreg,
          vf, vc,
          cand, clsl, regl, sem):
    f32 = jnp.float32
    i32 = jnp.int32
    wid = lax.axis_index("s") * 2 + lax.axis_index("c")

    copies = [pltpu.async_copy(fb, vf, sem),
              pltpu.async_copy(fc, vc, sem),
              pltpu.async_copy(base_h.at[wid], clsl, sem)]

    zero16 = jnp.zeros((16,), jnp.float32)

    def zbody(k, carry):
        for u in range(8):
            regl[pl.ds((k * 8 + u) * 16, 16)] = zero16
        return carry

    lax.fori_loop(0, _REG_W // 128, zbody, 0)
    for cdesc in copies:
        cdesc.wait()

    iota = jnp.arange(16, dtype=i32)
    widv = jnp.full((16,), wid, i32).astype(f32)
    cx_lo = -31.75 + 2.0 * widv
    cx_hi = cx_lo + 1.5

    def fbody(k, cnt):
        sl = pl.ds(k * 16, 16)
        bxv = vf[0, sl]
        bwv = vf[3, sl]
        blv = vf[4, sl]
        cv = vc[sl]
        wv = jnp.where(cv == 0, _W0, _W12)
        lv = jnp.where(cv == 0, _L0, jnp.where(cv == 1, _L1, _L2))
        thrv = jnp.where(cv == 0, _T0, _T12)
        hw = (bwv + wv) * 0.5
        m = (bxv + hw > cx_lo) & (bxv - hw < cx_hi)
        imax = jnp.minimum(bwv, wv) * jnp.minimum(blv, lv)
        m &= imax > (0.999 * thrv) * (wv * lv + bwv * blv - imax)
        cum = jnp.cumsum(m.astype(i32))
        plsc.store_scatter(cand, [cnt + cum - 1], k * 16 + iota, mask=m)
        return cnt + jnp.max(cum)

    ncand = lax.fori_loop(0, _NCHUNK, fbody, i32(0))

    def cbody(t, carry):
        b = plsc.load_gather(cand, [jnp.full((16,), t, i32)])
        zidx = jnp.zeros((16,), i32)
        bxv = plsc.load_gather(vf, [zidx, b])
        byv = plsc.load_gather(vf, [zidx + 1, b])
        bzv = plsc.load_gather(vf, [zidx + 2, b])
        bwv = plsc.load_gather(vf, [zidx + 3, b])
        blv = plsc.load_gather(vf, [zidx + 4, b])
        bhv = plsc.load_gather(vf, [zidx + 5, b])
        bywv = plsc.load_gather(vf, [zidx + 6, b])
        cv = plsc.load_gather(vc, [b])

        wv = jnp.where(cv == 0, _W0, _W12)
        lv = jnp.where(cv == 0, _L0, jnp.where(cv == 1, _L1, _L2))
        hv = jnp.where(cv == 0, _H0, _H12)
        czv = jnp.where(cv == 0, _CZ0, _CZ12)
        thrv = jnp.where(cv == 0, _T0, _T12)
        whv = wv * 0.5
        lhv = lv * 0.5
        area = wv * lv

        lo = (byv - (blv + lv) * 0.5 + 31.75) * 2.0
        jv = (lo + 16.0).astype(i32) - 15 + iota
        valid = (jv >= 0) & (jv < _NY)
        jsafe = jnp.clip(jv, 0, _NY - 1)
        cyv = -31.75 + 0.5 * jv.astype(f32)
        iyv = jnp.maximum(
            jnp.minimum(cyv + lhv, byv + blv * 0.5)
            - jnp.maximum(cyv - lhv, byv - blv * 0.5), 0.0)

        r1 = byv - cyv
        r2 = bzv - czv
        r3 = (bwv - wv) / wv
        r4 = (blv - lv) / lv
        r5 = (bhv - hv) / hv
        r6 = bywv - _YAW1
        c0v = jnp.where(cv == 0, 1, 0).astype(i32)
        c1v = jnp.where(cv == 1, 1, 0).astype(i32)
        c2v = jnp.where(cv == 2, 1, 0).astype(i32)
        zi = jnp.zeros((16,), i32)
        clsbase = cv * (_ROWS * _NY * 5) + jsafe * 5
        regbase = cv * (_ROWS * _NY * 7) + jsafe * 7

        for il in range(_ROWS):
            cxv = cx_lo + (0.5 * il)
            ixv = jnp.maximum(
                jnp.minimum(cxv + whv, bxv + bwv * 0.5)
                - jnp.maximum(cxv - whv, bxv - bwv * 0.5), 0.0)
            inter = ixv * iyv
            union = area + bwv * blv - inter
            iou = inter / jnp.maximum(union, 1e-6)
            posm = (iou > thrv) & valid

            @pl.when(jnp.any(posm))
            def _write(il=il, posm=posm, cxv=cxv):
                cb = clsbase + il * (_NY * 5)
                plsc.store_scatter(clsl, [cb], c0v, mask=posm)
                plsc.store_scatter(clsl, [cb + 1], c1v, mask=posm)
                plsc.store_scatter(clsl, [cb + 2], c2v, mask=posm)
                plsc.store_scatter(clsl, [cb + 3], zi, mask=posm)
                plsc.store_scatter(clsl, [cb + 4], zi, mask=posm)
                rb = regbase + il * (_NY * 7)
                plsc.store_scatter(regl, [rb], bxv - cxv, mask=posm)
                plsc.store_scatter(regl, [rb + 1], r1, mask=posm)
                plsc.store_scatter(regl, [rb + 2], r2, mask=posm)
                plsc.store_scatter(regl, [rb + 3], r3, mask=posm)
                plsc.store_scatter(regl, [rb + 4], r4, mask=posm)
                plsc.store_scatter(regl, [rb + 5], r5, mask=posm)
                plsc.store_scatter(regl, [rb + 6], r6, mask=posm)
        return carry

    lax.fori_loop(0, ncand, cbody, 0)

    pltpu.sync_copy(clsl, ocls.at[wid])
    pltpu.sync_copy(regl, oreg.at[wid])


_assign = pl.kernel(
    _body,
    out_type=(jax.ShapeDtypeStruct((_NW, _CLS_W), jnp.int32),
              jax.ShapeDtypeStruct((_NW, _REG_W), jnp.float32)),
    mesh=plsc.VectorSubcoreMesh(core_axis_name="c", subcore_axis_name="s",
                                num_cores=2, num_subcores=16),
    compiler_params=pltpu.CompilerParams(needs_layout_passes=False),
    scratch_types=[
        pltpu.VMEM((7, _NBP), jnp.float32),
        pltpu.VMEM((_NBP,), jnp.int32),
        pltpu.VMEM((1024,), jnp.int32),
        pltpu.VMEM((_CLS_W,), jnp.int32),
        pltpu.VMEM((_REG_W,), jnp.float32),
        pltpu.SemaphoreType.DMA,
    ],
)


def kernel(boxes, class_idx):
    f32 = jnp.float32
    pad = jnp.full((7, _NBP - _NB), 1.0, f32).at[0].set(1e9)
    fb = jnp.concatenate([boxes.T, pad], axis=1)
    fc = jnp.concatenate([class_idx.astype(jnp.int32),
                          jnp.zeros((_NBP - _NB,), jnp.int32)])
    base = jnp.asarray(_get_neg_base())
    cls_f, reg_f = _assign(fb, fc, base)
    cls = (cls_f.reshape(_NW, _NCLS, _ROWS, _NY, 5)
           .transpose(1, 0, 2, 3, 4).reshape(_NCLS, _NX, _NY, 5))
    reg = (reg_f.reshape(_NW, _NCLS, _ROWS, _NY, 7)
           .transpose(1, 0, 2, 3, 4).reshape(_NCLS, _NX, _NY, 7))
    return cls, reg
